# Initial kernel scaffold; baseline (speedup 1.0000x reference)
import jax
import jax.numpy as jnp
from jax.experimental import pallas as pl


def kernel(convFM_w, convFM_b, resMM_w1, resMM_b1, resMM_w2, resMM_b2, rf2_convFS_w, rf2_convFS_b, rf2_resFS_w1, rf2_resFS_b1, rf2_resFS_w2, rf2_resFS_b2, rf2_resMM_w1, rf2_resMM_b1, rf2_resMM_w2, rf2_resMM_b2, rf1_convFS_w, rf1_convFS_b, rf1_resFS_w1, rf1_resFS_b1, rf1_resFS_w2, rf1_resFS_b2, rf1_resMM_w1, rf1_resMM_b1, rf1_resMM_w2, rf1_resMM_b2, pred_w, pred_b, opt_feat, r2, r1):
    raise NotImplementedError("write your pallas kernel here")


# speedup vs baseline: 1.1620x; 1.1620x over previous
import functools
import math

import numpy as np

import jax
import jax.numpy as jnp
from jax.experimental import pallas as pl
from jax.experimental.pallas import tpu as pltpu


def _interp_matrix_np(out_size, in_size):
    scale = in_size / out_size
    i = np.arange(out_size, dtype=np.float64)
    src = np.maximum((i + 0.5) * scale - 0.5, 0.0)
    i0 = np.minimum(np.floor(src).astype(np.int64), in_size - 1)
    i1 = np.minimum(i0 + 1, in_size - 1)
    frac = src - i0
    m = np.zeros((out_size, in_size), dtype=np.float64)
    m[np.arange(out_size), i0] += 1.0 - frac
    m[np.arange(out_size), i1] += frac
    return m.astype(np.float32)


@functools.lru_cache(maxsize=None)
def _upsample_kron_np(hout, wout, hin, win):
    a = _interp_matrix_np(hout, hin)
    b = _interp_matrix_np(wout, win)
    k = np.einsum('ia,jb->ijab', a, b).reshape(hout * wout, hin * win)
    return k.astype(np.float32)


def _conv(x2d, pad_ref, w_ref, b_ref, H, W, relu_in):
    cin = w_ref.shape[0] // 9
    xin = jnp.maximum(x2d, 0.0) if relu_in else x2d
    pad_ref[1:H + 1, 1:W + 1, :cin] = xin.reshape(H, W, cin).astype(jnp.bfloat16)
    xp = pad_ref[...]
    col = jnp.concatenate(
        [xp[dy:dy + H, dx:dx + W, :cin].reshape(H * W, cin)
         for dy in range(3) for dx in range(3)], axis=1)
    out = jnp.dot(col, w_ref[...], preferred_element_type=jnp.float32)
    return out + b_ref[...]


def _zero_border(pad_ref, H, W):
    c = pad_ref.shape[-1]
    z_row = jnp.zeros((1, W + 2, c), jnp.bfloat16)
    z_col = jnp.zeros((H + 2, 1, c), jnp.bfloat16)
    pad_ref[0:1, :, :] = z_row
    pad_ref[H + 1:H + 2, :, :] = z_row
    pad_ref[:, 0:1, :] = z_col
    pad_ref[:, W + 1:W + 2, :] = z_col


def _head_kernel(f_ref, w0, b0, w1, b1, w2, b2, out_ref, pad_ref, *, H, W):
    _zero_border(pad_ref, H, W)
    x = _conv(f_ref[...], pad_ref, w0, b0, H, W, relu_in=False)
    r = _conv(x, pad_ref, w1, b1, H, W, relu_in=True)
    r = _conv(r, pad_ref, w2, b2, H, W, relu_in=True)
    out_ref[...] = (x + r).astype(jnp.bfloat16)


def _head_stage(f, w0, b0, w1, b1, w2, b2, H, W):
    N, hw, c_f = f.shape
    c_out = w0.shape[1]
    args = [f, w0, b0, w1, b1, w2, b2]
    in_specs = [pl.BlockSpec((None, hw, c_f), lambda n: (n, 0, 0))]
    in_specs += [pl.BlockSpec(a.shape, lambda n: (0, 0)) for a in args[1:]]
    return pl.pallas_call(
        functools.partial(_head_kernel, H=H, W=W),
        out_shape=jax.ShapeDtypeStruct((N, hw, c_out), jnp.bfloat16),
        grid=(N,),
        in_specs=in_specs,
        out_specs=pl.BlockSpec((None, hw, c_out), lambda n: (n, 0, 0)),
        scratch_shapes=[pltpu.VMEM((H + 2, W + 2, max(c_f, c_out)), jnp.bfloat16)],
        compiler_params=pltpu.CompilerParams(dimension_semantics=("parallel",)),
    )(*args)


def _refine_kernel(*refs, H, W, with_pred):
    if with_pred:
        (f_ref, mp_ref, kr_ref, wfs, bfs, w11, b11, w12, b12,
         w21, b21, w22, b22, wd, bd, out_ref, pad_ref) = refs
    else:
        (f_ref, mp_ref, kr_ref, wfs, bfs, w11, b11, w12, b12,
         w21, b21, w22, b22, out_ref, pad_ref) = refs
    _zero_border(pad_ref, H, W)
    up = jnp.dot(kr_ref[...], mp_ref[...], preferred_element_type=jnp.float32)
    x = _conv(f_ref[...], pad_ref, wfs, bfs, H, W, relu_in=False)
    r = _conv(x, pad_ref, w11, b11, H, W, relu_in=True)
    r = _conv(r, pad_ref, w12, b12, H, W, relu_in=True)
    m = x + r + up
    r = _conv(m, pad_ref, w21, b21, H, W, relu_in=True)
    m = m + _conv(r, pad_ref, w22, b22, H, W, relu_in=True)
    if with_pred:
        d = _conv(m, pad_ref, wd, bd, H, W, relu_in=True)
        out_ref[...] = d.reshape(1, H * W)
    else:
        out_ref[...] = m.astype(jnp.bfloat16)


def _refine_stage(f, mprev, kr, wfs, bfs, w11, b11, w12, b12, w21, b21,
                  w22, b22, H, W, wd=None, bd=None):
    N, hw, c_f = f.shape
    c_out = wfs.shape[1]
    with_pred = wd is not None
    args = [f, mprev, kr, wfs, bfs, w11, b11, w12, b12, w21, b21, w22, b22]
    if with_pred:
        args += [wd, bd]
        out_shape = jax.ShapeDtypeStruct((N, 1, hw), jnp.float32)
        out_spec = pl.BlockSpec((None, 1, hw), lambda n: (n, 0, 0))
    else:
        out_shape = jax.ShapeDtypeStruct((N, hw, c_out), jnp.bfloat16)
        out_spec = pl.BlockSpec((None, hw, c_out), lambda n: (n, 0, 0))
    in_specs = [pl.BlockSpec((None, hw, c_f), lambda n: (n, 0, 0)),
                pl.BlockSpec((None,) + mprev.shape[1:], lambda n: (n, 0, 0))]
    in_specs += [pl.BlockSpec(a.shape, lambda n: (0, 0)) for a in args[2:]]
    return pl.pallas_call(
        functools.partial(_refine_kernel, H=H, W=W, with_pred=with_pred),
        out_shape=out_shape,
        grid=(N,),
        in_specs=in_specs,
        out_specs=out_spec,
        scratch_shapes=[pltpu.VMEM((H + 2, W + 2, max(c_f, c_out)), jnp.bfloat16)],
        compiler_params=pltpu.CompilerParams(dimension_semantics=("parallel",)),
    )(*args)


def _agg_kernel(d_ref, a4_ref, bt_ref, out_ref, *, no, max_obj, hf, wf):
    eps = 1e-7

    def logit(e):
        e = jnp.clip(e, eps, 1.0 - eps)
        return jnp.log(e / (1.0 - e))

    t = jnp.dot(a4_ref[...], d_ref[...], preferred_element_type=jnp.float32)
    u = jnp.dot(t, bt_ref[...], preferred_element_type=jnp.float32)
    ps = jax.nn.sigmoid(u)
    rows = [ps[i * hf:(i + 1) * hf] for i in range(no)]
    bg = 1.0 - rows[0]
    for i in range(1, no):
        bg = bg * (1.0 - rows[i])
    out_ref[0:1] = logit(bg)[None]
    for i in range(no):
        out_ref[1 + i:2 + i] = logit(rows[i])[None]
    pad_val = math.log(eps / (1.0 - eps))
    for j in range(no + 1, max_obj):
        out_ref[j:j + 1] = jnp.full((1, hf, wf), pad_val, jnp.float32)


def _aggregate(d3, a4, bt, no, max_obj, hf, wf):
    bs = d3.shape[0]
    args = [d3, a4, bt]
    in_specs = [pl.BlockSpec((None,) + d3.shape[1:], lambda n: (n, 0, 0)),
                pl.BlockSpec(a4.shape, lambda n: (0, 0)),
                pl.BlockSpec(bt.shape, lambda n: (0, 0))]
    return pl.pallas_call(
        functools.partial(_agg_kernel, no=no, max_obj=max_obj, hf=hf, wf=wf),
        out_shape=jax.ShapeDtypeStruct((bs, max_obj, hf, wf), jnp.float32),
        grid=(bs,),
        in_specs=in_specs,
        out_specs=pl.BlockSpec((None, max_obj, hf, wf), lambda n: (n, 0, 0, 0)),
        compiler_params=pltpu.CompilerParams(dimension_semantics=("parallel",)),
    )(*args)


def kernel(convFM_w, convFM_b, resMM_w1, resMM_b1, resMM_w2, resMM_b2,
           rf2_convFS_w, rf2_convFS_b, rf2_resFS_w1, rf2_resFS_b1,
           rf2_resFS_w2, rf2_resFS_b2, rf2_resMM_w1, rf2_resMM_b1,
           rf2_resMM_w2, rf2_resMM_b2, rf1_convFS_w, rf1_convFS_b,
           rf1_resFS_w1, rf1_resFS_b1, rf1_resFS_w2, rf1_resFS_b2,
           rf1_resMM_w1, rf1_resMM_b1, rf1_resMM_w2, rf1_resMM_b2,
           pred_w, pred_b, opt_feat, r2, r1):
    bs, max_obj, h, w = 4, 5, 256, 256
    N, c_in, H3, W3 = opt_feat.shape
    _, c_in_2, H2, W2 = r2.shape
    _, c_in_1, H1, W1 = r1.shape
    no = N // bs

    bf16 = jnp.bfloat16
    cast = lambda a: a.astype(bf16)

    f3 = jnp.transpose(opt_feat.reshape(N, c_in, H3 * W3), (0, 2, 1)).astype(bf16)
    f2 = jnp.transpose(r2.reshape(N, c_in_2, H2 * W2), (0, 2, 1)).astype(bf16)
    f1 = jnp.transpose(r1.reshape(N, c_in_1, H1 * W1), (0, 2, 1)).astype(bf16)

    m3 = _head_stage(f3, cast(convFM_w), convFM_b, cast(resMM_w1), resMM_b1,
                     cast(resMM_w2), resMM_b2, H3, W3)

    kr2 = jnp.asarray(_upsample_kron_np(H2, W2, H3, W3), bf16)
    m2 = _refine_stage(f2, m3, kr2,
                       cast(rf2_convFS_w), rf2_convFS_b,
                       cast(rf2_resFS_w1), rf2_resFS_b1,
                       cast(rf2_resFS_w2), rf2_resFS_b2,
                       cast(rf2_resMM_w1), rf2_resMM_b1,
                       cast(rf2_resMM_w2), rf2_resMM_b2, H2, W2)

    wd = cast(pred_w[:, 1:2] - pred_w[:, 0:1])
    bd = (pred_b[:, 1:2] - pred_b[:, 0:1]).astype(jnp.float32)
    kr1 = jnp.asarray(_upsample_kron_np(H1, W1, H2, W2), bf16)
    d = _refine_stage(f1, m2, kr1,
                      cast(rf1_convFS_w), rf1_convFS_b,
                      cast(rf1_resFS_w1), rf1_resFS_b1,
                      cast(rf1_resFS_w2), rf1_resFS_b2,
                      cast(rf1_resMM_w1), rf1_resMM_b1,
                      cast(rf1_resMM_w2), rf1_resMM_b2, H1, W1,
                      wd=wd, bd=bd)

    d3 = d.reshape(bs, no * H1, W1)
    a4 = jnp.asarray(np.kron(np.eye(no, dtype=np.float32),
                             _interp_matrix_np(h, H1)), jnp.float32)
    bt = jnp.asarray(_interp_matrix_np(w, W1).T, jnp.float32)
    return _aggregate(d3, a4, bt, no, max_obj, h, w)
